# Initial kernel scaffold; baseline (speedup 1.0000x reference)
import jax
import jax.numpy as jnp
from jax.experimental import pallas as pl


def kernel(x, edge_index, W1, b1, W2, b2, gamma, beta):
    raise NotImplementedError("write your pallas kernel here")


# speedup vs baseline: 5.8399x; 5.8399x over previous
import functools

import jax
import jax.numpy as jnp
from jax import lax
from jax.experimental import pallas as pl
from jax.experimental.pallas import tpu as pltpu
from jax.experimental.pallas import tpu_sc as plsc

_NC = 2
_NS = 16
_NW = _NC * _NS
_CHUNK = 128


def _segment_sum_partials(x, edge_index):
    n, d = x.shape
    e = edge_index.shape[1]
    assert e % _CHUNK == 0 and d % 16 == 0
    n_chunks = e // _CHUNK
    row_step = (n // _NS) // 8 * 8
    row_win = n - row_step * (_NS - 1)
    assert row_win % 8 == 0 and row_win >= row_step

    mesh = plsc.VectorSubcoreMesh(core_axis_name="c", subcore_axis_name="s")

    @functools.partial(
        pl.kernel,
        mesh=mesh,
        out_type=jax.ShapeDtypeStruct((_NC, n, d), jnp.float32),
        scratch_types=[
            pltpu.VMEM((2, _CHUNK), jnp.int32),
            pltpu.VMEM((_CHUNK, d), jnp.float32),
            pltpu.VMEM_SHARED((n, d), jnp.float32),
        ],
    )
    def seg_kernel(x_hbm, ei_hbm, out_hbm, idx_v, rows_v, agg_sh):
        cid = lax.axis_index("c")
        sid = lax.axis_index("s")
        wid = sid * _NC + cid

        zvec = jnp.zeros((16,), jnp.float32)

        def zero_row(r, carry):
            for c in range(d // 16):
                rows_v[r, pl.ds(c * 16, 16)] = zvec
            return carry

        lax.fori_loop(0, _CHUNK, zero_row, 0)
        row0 = sid * row_step
        full = row_win // _CHUNK
        for k in range(full):
            pltpu.sync_copy(rows_v, agg_sh.at[pl.ds(row0 + k * _CHUNK, _CHUNK)])
        rem = row_win - full * _CHUNK
        if rem:
            pltpu.sync_copy(rows_v.at[pl.ds(0, rem)],
                            agg_sh.at[pl.ds(row0 + full * _CHUNK, rem)])
        plsc.subcore_barrier()

        n_base = n_chunks // _NW
        n_rem = n_chunks % _NW
        n_my = n_base + (wid < n_rem).astype(jnp.int32)

        def body(t, carry):
            base = (wid + t * _NW) * _CHUNK
            pltpu.sync_copy(ei_hbm.at[:, pl.ds(base, _CHUNK)], idx_v)
            pltpu.sync_copy(x_hbm.at[idx_v.at[0]], rows_v)
            pltpu.sync_copy(rows_v, agg_sh.at[idx_v.at[1]], add=True)
            return carry

        lax.fori_loop(0, n_my, body, 0)
        plsc.subcore_barrier()

        pltpu.sync_copy(agg_sh.at[pl.ds(row0, row_win)],
                        out_hbm.at[cid, pl.ds(row0, row_win)])

    return seg_kernel(x, edge_index)


_BLK = 1000


def _mlp_layer(x, p0, p1, w1, b1, w2, b2, g, bt):
    n, d = x.shape
    assert n % _BLK == 0
    inv_std = float(1.0 / (1.0 + 1e-5) ** 0.5)

    def body(x_ref, p0_ref, p1_ref, w1_ref, b1_ref, w2_ref, b2_ref, g_ref,
             bt_ref, o_ref):
        h = x_ref[...] + p0_ref[...] + p1_ref[...]
        h = lax.dot(h, w1_ref[...], precision=lax.Precision.HIGHEST,
                    preferred_element_type=jnp.float32) + b1_ref[...]
        h = jnp.maximum(h, 0.0)
        h = lax.dot(h, w2_ref[...], precision=lax.Precision.HIGHEST,
                    preferred_element_type=jnp.float32) + b2_ref[...]
        h = jnp.maximum(h, 0.0)
        o_ref[...] = g_ref[...] * (h * inv_std) + bt_ref[...]

    blk = pl.BlockSpec((_BLK, d), lambda i: (i, 0))
    wblk = pl.BlockSpec((d, d), lambda i: (0, 0))
    vblk = pl.BlockSpec((1, d), lambda i: (0, 0))
    return pl.pallas_call(
        body,
        grid=(n // _BLK,),
        in_specs=[blk, blk, blk, wblk, vblk, wblk, vblk, vblk, vblk],
        out_specs=blk,
        out_shape=jax.ShapeDtypeStruct((n, d), jnp.float32),
    )(x, p0, p1, w1, b1.reshape(1, d), w2, b2.reshape(1, d),
      g.reshape(1, d), bt.reshape(1, d))


def kernel(x, edge_index, W1, b1, W2, b2, gamma, beta):
    num_layers = W1.shape[0]
    out = x
    recs = []
    for i in range(num_layers):
        parts = _segment_sum_partials(out, edge_index)
        out = _mlp_layer(out, parts[0], parts[1], W1[i], b1[i], W2[i], b2[i],
                         gamma[i], beta[i])
        recs.append(out)
    return jnp.concatenate(recs, axis=-1)


# speedup vs baseline: 7.1549x; 1.2252x over previous
import functools

import jax
import jax.numpy as jnp
from jax import lax
from jax.experimental import pallas as pl
from jax.experimental.pallas import tpu as pltpu
from jax.experimental.pallas import tpu_sc as plsc

_NC = 2
_NS = 16
_NW = _NC * _NS
_CHUNK = 128
_NB = 3


def _segment_sum_partials(x, edge_index):
    n, d = x.shape
    e = edge_index.shape[1]
    assert e % _CHUNK == 0 and d % 16 == 0
    n_chunks = e // _CHUNK
    row_step = (n // _NS) // 8 * 8
    row_win = n - row_step * (_NS - 1)
    assert row_win % 8 == 0 and row_win >= row_step

    mesh = plsc.VectorSubcoreMesh(core_axis_name="c", subcore_axis_name="s")

    @functools.partial(
        pl.kernel,
        mesh=mesh,
        out_type=jax.ShapeDtypeStruct((_NC, n, d), jnp.float32),
        scratch_types=[
            pltpu.VMEM((_NB, 2, _CHUNK), jnp.int32),
            pltpu.VMEM((_NB, _CHUNK, d), jnp.float32),
            pltpu.VMEM_SHARED((n, d), jnp.float32),
            pltpu.SemaphoreType.DMA((_NB,)),
            pltpu.SemaphoreType.DMA((_NB,)),
        ],
    )
    def seg_kernel(x_hbm, ei_hbm, out_hbm, idx_v, rows_v, agg_sh, gsem, ssem):
        cid = lax.axis_index("c")
        sid = lax.axis_index("s")
        wid = sid * _NC + cid

        zvec = jnp.zeros((16,), jnp.float32)

        def zero_row(r, carry):
            for c in range(d // 16):
                rows_v[0, r, pl.ds(c * 16, 16)] = zvec
            return carry

        lax.fori_loop(0, _CHUNK, zero_row, 0)
        row0 = sid * row_step
        full = row_win // _CHUNK
        for k in range(full):
            pltpu.sync_copy(rows_v.at[0],
                            agg_sh.at[pl.ds(row0 + k * _CHUNK, _CHUNK)])
        rem = row_win - full * _CHUNK
        if rem:
            pltpu.sync_copy(rows_v.at[0, pl.ds(0, rem)],
                            agg_sh.at[pl.ds(row0 + full * _CHUNK, rem)])
        plsc.subcore_barrier()

        n_base = n_chunks // _NW
        n_rem = n_chunks % _NW
        n_my = n_base + (wid < n_rem).astype(jnp.int32)

        def load_idx(t, slot):
            base = (wid + t * _NW) * _CHUNK
            pltpu.sync_copy(ei_hbm.at[:, pl.ds(base, _CHUNK)], idx_v.at[slot])

        def gather(slot):
            return pltpu.make_async_copy(x_hbm.at[idx_v.at[slot, 0]],
                                         rows_v.at[slot], gsem.at[slot])

        def scatter(slot):
            return pltpu.make_async_copy(rows_v.at[slot],
                                         agg_sh.at[idx_v.at[slot, 1]],
                                         ssem.at[slot])

        load_idx(0, 0)
        gather(0).start()

        def group(g, carry):
            for b in range(_NB):
                t = g * _NB + b

                @pl.when(t < n_my)
                def _():
                    gather(b).wait()
                    scatter(b).start(add=True)
                nb = (b + 1) % _NB
                t1 = t + 1

                @pl.when(jnp.logical_and(t1 < n_my, t1 >= _NB))
                def _():
                    scatter(nb).wait()

                @pl.when(t1 < n_my)
                def _():
                    load_idx(t1, nb)
                    gather(nb).start()
            return carry

        lax.fori_loop(0, (n_my + _NB - 1) // _NB, group, 0)
        for b in range(_NB):
            scatter(b).wait()
        plsc.subcore_barrier()

        pltpu.sync_copy(agg_sh.at[pl.ds(row0, row_win)],
                        out_hbm.at[cid, pl.ds(row0, row_win)])

    return seg_kernel(x, edge_index)


_BLK = 1000


def _mlp_layer(x, p0, p1, w1, b1, w2, b2, g, bt):
    n, d = x.shape
    assert n % _BLK == 0
    inv_std = float(1.0 / (1.0 + 1e-5) ** 0.5)

    def body(x_ref, p0_ref, p1_ref, w1_ref, b1_ref, w2_ref, b2_ref, g_ref,
             bt_ref, o_ref):
        h = x_ref[...] + p0_ref[...] + p1_ref[...]
        h = lax.dot(h, w1_ref[...], precision=lax.Precision.HIGHEST,
                    preferred_element_type=jnp.float32) + b1_ref[...]
        h = jnp.maximum(h, 0.0)
        h = lax.dot(h, w2_ref[...], precision=lax.Precision.HIGHEST,
                    preferred_element_type=jnp.float32) + b2_ref[...]
        h = jnp.maximum(h, 0.0)
        o_ref[...] = g_ref[...] * (h * inv_std) + bt_ref[...]

    blk = pl.BlockSpec((_BLK, d), lambda i: (i, 0))
    wblk = pl.BlockSpec((d, d), lambda i: (0, 0))
    vblk = pl.BlockSpec((1, d), lambda i: (0, 0))
    return pl.pallas_call(
        body,
        grid=(n // _BLK,),
        in_specs=[blk, blk, blk, wblk, vblk, wblk, vblk, vblk, vblk],
        out_specs=blk,
        out_shape=jax.ShapeDtypeStruct((n, d), jnp.float32),
    )(x, p0, p1, w1, b1.reshape(1, d), w2, b2.reshape(1, d),
      g.reshape(1, d), bt.reshape(1, d))


def kernel(x, edge_index, W1, b1, W2, b2, gamma, beta):
    num_layers = W1.shape[0]
    out = x
    recs = []
    for i in range(num_layers):
        parts = _segment_sum_partials(out, edge_index)
        out = _mlp_layer(out, parts[0], parts[1], W1[i], b1[i], W2[i], b2[i],
                         gamma[i], beta[i])
        recs.append(out)
    return jnp.concatenate(recs, axis=-1)


# speedup vs baseline: 8.6636x; 1.2109x over previous
import functools

import jax
import jax.numpy as jnp
from jax import lax
from jax.experimental import pallas as pl
from jax.experimental.pallas import tpu as pltpu
from jax.experimental.pallas import tpu_sc as plsc

_NC = 2
_NS = 16
_NW = _NC * _NS
_CHUNK = 128
_NB = 3
_NI = 6


def _segment_sum_partials(x, edge_index):
    n, d = x.shape
    e = edge_index.shape[1]
    assert e % _CHUNK == 0 and d % 16 == 0
    n_chunks = e // _CHUNK
    row_step = (n // _NS) // 8 * 8
    row_win = n - row_step * (_NS - 1)
    assert row_win % 8 == 0 and row_win >= row_step

    mesh = plsc.VectorSubcoreMesh(core_axis_name="c", subcore_axis_name="s")

    @functools.partial(
        pl.kernel,
        mesh=mesh,
        out_type=jax.ShapeDtypeStruct((_NC, n, d), jnp.float32),
        scratch_types=[
            pltpu.VMEM((_NI, 2, _CHUNK), jnp.int32),
            pltpu.VMEM((_NB, _CHUNK, d), jnp.float32),
            pltpu.VMEM_SHARED((n, d), jnp.float32),
            pltpu.SemaphoreType.DMA((_NI,)),
            pltpu.SemaphoreType.DMA((_NB,)),
            pltpu.SemaphoreType.DMA((_NB,)),
        ],
    )
    def seg_kernel(x_hbm, ei_hbm, out_hbm, idx_v, rows_v, agg_sh,
                   isem, gsem, ssem):
        cid = lax.axis_index("c")
        sid = lax.axis_index("s")
        wid = sid * _NC + cid

        zvec = jnp.zeros((16,), jnp.float32)

        def zero_row(r, carry):
            for c in range(d // 16):
                rows_v[0, r, pl.ds(c * 16, 16)] = zvec
            return carry

        c0 = (n_chunks * wid) // _NW
        n_my = (n_chunks * (wid + 1)) // _NW - c0

        lax.fori_loop(0, _CHUNK, zero_row, 0)
        row0 = sid * row_step
        full = row_win // _CHUNK
        for k in range(full):
            pltpu.sync_copy(rows_v.at[0],
                            agg_sh.at[pl.ds(row0 + k * _CHUNK, _CHUNK)])
        rem = row_win - full * _CHUNK
        if rem:
            pltpu.sync_copy(rows_v.at[0, pl.ds(0, rem)],
                            agg_sh.at[pl.ds(row0 + full * _CHUNK, rem)])
        plsc.subcore_barrier()

        def load_idx(t, islot):
            return pltpu.make_async_copy(
                ei_hbm.at[:, pl.ds((c0 + t) * _CHUNK, _CHUNK)],
                idx_v.at[islot], isem.at[islot])

        def gather(bslot, islot):
            return pltpu.make_async_copy(x_hbm.at[idx_v.at[islot, 0]],
                                         rows_v.at[bslot], gsem.at[bslot])

        def scatter(bslot, islot):
            return pltpu.make_async_copy(rows_v.at[bslot],
                                         agg_sh.at[idx_v.at[islot, 1]],
                                         ssem.at[bslot])

        for j in range(_NI // 2):
            load_idx(j, j).start()
        load_idx(0, 0).wait()
        gather(0, 0).start()

        def group(g, carry):
            for u in range(_NI):
                t = g * _NI + u
                b = u % _NB
                ib = u % _NI

                @pl.when(t < n_my)
                def _():
                    gather(b, ib).wait()
                    scatter(b, ib).start(add=True)
                t1 = t + 1
                b1 = (u + 1) % _NB
                ib1 = (u + 1) % _NI

                @pl.when(jnp.logical_and(t1 < n_my, t1 >= _NB))
                def _():
                    scatter(b1, ib1).wait()
                t3 = t + _NI // 2
                ib3 = (u + _NI // 2) % _NI

                @pl.when(t3 < n_my)
                def _():
                    load_idx(t3, ib3).start()

                @pl.when(t1 < n_my)
                def _():
                    load_idx(t1, ib1).wait()
                    gather(b1, ib1).start()
            return carry

        lax.fori_loop(0, (n_my + _NI - 1) // _NI, group, 0)
        for b in range(_NB):
            scatter(b, b).wait()
        plsc.subcore_barrier()

        pltpu.sync_copy(agg_sh.at[pl.ds(row0, row_win)],
                        out_hbm.at[cid, pl.ds(row0, row_win)])

    return seg_kernel(x, edge_index)


_BLK = 1000


def _mlp_layer(x, p0, p1, w1, b1, w2, b2, g, bt):
    n, d = x.shape
    assert n % _BLK == 0
    inv_std = float(1.0 / (1.0 + 1e-5) ** 0.5)

    def body(x_ref, p0_ref, p1_ref, w1_ref, b1_ref, w2_ref, b2_ref, g_ref,
             bt_ref, o_ref):
        h = x_ref[...] + p0_ref[...] + p1_ref[...]
        h = lax.dot(h, w1_ref[...], precision=lax.Precision.HIGHEST,
                    preferred_element_type=jnp.float32) + b1_ref[...]
        h = jnp.maximum(h, 0.0)
        h = lax.dot(h, w2_ref[...], precision=lax.Precision.HIGHEST,
                    preferred_element_type=jnp.float32) + b2_ref[...]
        h = jnp.maximum(h, 0.0)
        o_ref[...] = g_ref[...] * (h * inv_std) + bt_ref[...]

    blk = pl.BlockSpec((_BLK, d), lambda i: (i, 0))
    wblk = pl.BlockSpec((d, d), lambda i: (0, 0))
    vblk = pl.BlockSpec((1, d), lambda i: (0, 0))
    return pl.pallas_call(
        body,
        grid=(n // _BLK,),
        in_specs=[blk, blk, blk, wblk, vblk, wblk, vblk, vblk, vblk],
        out_specs=blk,
        out_shape=jax.ShapeDtypeStruct((n, d), jnp.float32),
    )(x, p0, p1, w1, b1.reshape(1, d), w2, b2.reshape(1, d),
      g.reshape(1, d), bt.reshape(1, d))


def kernel(x, edge_index, W1, b1, W2, b2, gamma, beta):
    num_layers = W1.shape[0]
    out = x
    recs = []
    for i in range(num_layers):
        parts = _segment_sum_partials(out, edge_index)
        out = _mlp_layer(out, parts[0], parts[1], W1[i], b1[i], W2[i], b2[i],
                         gamma[i], beta[i])
        recs.append(out)
    return jnp.concatenate(recs, axis=-1)


# speedup vs baseline: 10.2978x; 1.1886x over previous
import functools

import jax
import jax.numpy as jnp
from jax import lax
from jax.experimental import pallas as pl
from jax.experimental.pallas import tpu as pltpu
from jax.experimental.pallas import tpu_sc as plsc

_NC = 2
_NS = 16
_NW = _NC * _NS
_CHUNK = 128
_NB = 3
_NI = 6


def _segment_sum_partials(x, edge_index):
    n, d = x.shape
    e = edge_index.shape[1]
    assert e % _CHUNK == 0 and d % 16 == 0
    n_chunks = e // _CHUNK
    row_step = (n // _NS) // 8 * 8
    row_win = n - row_step * (_NS - 1)
    assert row_win % 8 == 0 and row_win >= row_step

    mesh = plsc.VectorSubcoreMesh(core_axis_name="c", subcore_axis_name="s")

    @functools.partial(
        pl.kernel,
        mesh=mesh,
        out_type=[jax.ShapeDtypeStruct((n, d), jnp.float32),
                  jax.ShapeDtypeStruct((n, d), jnp.float32)],
        scratch_types=[
            pltpu.VMEM((_NI, 2, _CHUNK), jnp.int32),
            pltpu.VMEM((_NB, _CHUNK, d), jnp.float32),
            pltpu.VMEM_SHARED((n, d), jnp.float32),
            pltpu.SemaphoreType.DMA((_NI,)),
            pltpu.SemaphoreType.DMA((_NB,)),
            pltpu.SemaphoreType.DMA((_NB,)),
        ],
    )
    def seg_kernel(x_hbm, ei_hbm, out0_hbm, out1_hbm, idx_v, rows_v, agg_sh,
                   isem, gsem, ssem):
        cid = lax.axis_index("c")
        sid = lax.axis_index("s")
        wid = sid * _NC + cid

        zvec = jnp.zeros((16,), jnp.float32)

        def zero_row(r, carry):
            for c in range(d // 16):
                rows_v[0, r, pl.ds(c * 16, 16)] = zvec
            return carry

        c0 = (n_chunks * wid) // _NW
        n_my = (n_chunks * (wid + 1)) // _NW - c0

        lax.fori_loop(0, _CHUNK, zero_row, 0)
        row0 = sid * row_step
        full = row_win // _CHUNK
        for k in range(full):
            pltpu.sync_copy(rows_v.at[0],
                            agg_sh.at[pl.ds(row0 + k * _CHUNK, _CHUNK)])
        rem = row_win - full * _CHUNK
        if rem:
            pltpu.sync_copy(rows_v.at[0, pl.ds(0, rem)],
                            agg_sh.at[pl.ds(row0 + full * _CHUNK, rem)])
        plsc.subcore_barrier()

        def load_idx(t, islot):
            return pltpu.make_async_copy(
                ei_hbm.at[:, pl.ds((c0 + t) * _CHUNK, _CHUNK)],
                idx_v.at[islot], isem.at[islot])

        def gather(bslot, islot):
            return pltpu.make_async_copy(x_hbm.at[idx_v.at[islot, 0]],
                                         rows_v.at[bslot], gsem.at[bslot])

        def scatter(bslot, islot):
            return pltpu.make_async_copy(rows_v.at[bslot],
                                         agg_sh.at[idx_v.at[islot, 1]],
                                         ssem.at[bslot])

        for j in range(_NI // 2):
            load_idx(j, j).start()
        load_idx(0, 0).wait()
        gather(0, 0).start()

        def group(g, carry):
            for u in range(_NI):
                t = g * _NI + u
                b = u % _NB
                ib = u % _NI

                @pl.when(t < n_my)
                def _():
                    gather(b, ib).wait()
                    scatter(b, ib).start(add=True)
                t1 = t + 1
                b1 = (u + 1) % _NB
                ib1 = (u + 1) % _NI

                @pl.when(jnp.logical_and(t1 < n_my, t1 >= _NB))
                def _():
                    scatter(b1, ib1).wait()
                t3 = t + _NI // 2
                ib3 = (u + _NI // 2) % _NI

                @pl.when(t3 < n_my)
                def _():
                    load_idx(t3, ib3).start()

                @pl.when(t1 < n_my)
                def _():
                    load_idx(t1, ib1).wait()
                    gather(b1, ib1).start()
            return carry

        lax.fori_loop(0, (n_my + _NI - 1) // _NI, group, 0)
        for b in range(_NB):
            scatter(b, b).wait()
        plsc.subcore_barrier()

        @pl.when(cid == 0)
        def _():
            pltpu.sync_copy(agg_sh.at[pl.ds(row0, row_win)],
                            out0_hbm.at[pl.ds(row0, row_win)])

        @pl.when(cid == 1)
        def _():
            pltpu.sync_copy(agg_sh.at[pl.ds(row0, row_win)],
                            out1_hbm.at[pl.ds(row0, row_win)])

    return seg_kernel(x, edge_index)


_BLK = 1000


def _mlp_layer(x, p0, p1, w1, b1, w2, b2, g, bt):
    n, d = x.shape
    assert n % _BLK == 0
    inv_std = float(1.0 / (1.0 + 1e-5) ** 0.5)

    def body(x_ref, p0_ref, p1_ref, w1_ref, b1_ref, w2_ref, b2_ref, g_ref,
             bt_ref, o_ref):
        h = x_ref[...] + p0_ref[...] + p1_ref[...]
        h = lax.dot(h, w1_ref[...],
                    preferred_element_type=jnp.float32) + b1_ref[...]
        h = jnp.maximum(h, 0.0)
        h = lax.dot(h, w2_ref[...],
                    preferred_element_type=jnp.float32) + b2_ref[...]
        h = jnp.maximum(h, 0.0)
        o_ref[...] = g_ref[...] * (h * inv_std) + bt_ref[...]

    blk = pl.BlockSpec((_BLK, d), lambda i: (i, 0))
    wblk = pl.BlockSpec((d, d), lambda i: (0, 0))
    vblk = pl.BlockSpec((1, d), lambda i: (0, 0))
    return pl.pallas_call(
        body,
        grid=(n // _BLK,),
        in_specs=[blk, blk, blk, wblk, vblk, wblk, vblk, vblk, vblk],
        out_specs=blk,
        out_shape=jax.ShapeDtypeStruct((n, d), jnp.float32),
    )(x, p0, p1, w1, b1.reshape(1, d), w2, b2.reshape(1, d),
      g.reshape(1, d), bt.reshape(1, d))


def kernel(x, edge_index, W1, b1, W2, b2, gamma, beta):
    num_layers = W1.shape[0]
    out = x
    recs = []
    for i in range(num_layers):
        p0, p1 = _segment_sum_partials(out, edge_index)
        out = _mlp_layer(out, p0, p1, W1[i], b1[i], W2[i], b2[i],
                         gamma[i], beta[i])
        recs.append(out)
    return jnp.concatenate(recs, axis=-1)


# speedup vs baseline: 10.5233x; 1.0219x over previous
import functools

import jax
import jax.numpy as jnp
from jax import lax
from jax.experimental import pallas as pl
from jax.experimental.pallas import tpu as pltpu
from jax.experimental.pallas import tpu_sc as plsc

_NC = 2
_NS = 16
_NW = _NC * _NS
_CHUNK = 128
_NB = 3
_NI = 6


def _segment_sum_partials(x, edge_index):
    n, d = x.shape
    e = edge_index.shape[1]
    assert e % _CHUNK == 0 and d % 16 == 0
    n_chunks = e // _CHUNK
    row_step = (n // _NS) // 8 * 8
    row_win = n - row_step * (_NS - 1)
    assert row_win % 8 == 0 and row_win >= row_step

    mesh = plsc.VectorSubcoreMesh(core_axis_name="c", subcore_axis_name="s")

    @functools.partial(
        pl.kernel,
        mesh=mesh,
        out_type=[jax.ShapeDtypeStruct((n, d), jnp.float32),
                  jax.ShapeDtypeStruct((n, d), jnp.float32)],
        scratch_types=[
            pltpu.VMEM((_NI, 2, _CHUNK), jnp.int32),
            pltpu.VMEM((_NB, _CHUNK, d), jnp.float32),
            pltpu.VMEM_SHARED((n, d), jnp.float32),
            pltpu.SemaphoreType.DMA((_NI,)),
            pltpu.SemaphoreType.DMA((_NB,)),
            pltpu.SemaphoreType.DMA((_NB,)),
        ],
    )
    def seg_kernel(x_hbm, ei_hbm, out0_hbm, out1_hbm, idx_v, rows_v, agg_sh,
                   isem, gsem, ssem):
        cid = lax.axis_index("c")
        sid = lax.axis_index("s")
        wid = sid * _NC + cid

        zvec = jnp.zeros((16,), jnp.float32)

        def zero_row(r, carry):
            for c in range(d // 16):
                rows_v[0, r, pl.ds(c * 16, 16)] = zvec
            return carry

        c0 = (n_chunks * wid) // _NW
        n_my = (n_chunks * (wid + 1)) // _NW - c0

        lax.fori_loop(0, _CHUNK, zero_row, 0)
        row0 = sid * row_step
        full = row_win // _CHUNK
        for k in range(full):
            pltpu.sync_copy(rows_v.at[0],
                            agg_sh.at[pl.ds(row0 + k * _CHUNK, _CHUNK)])
        rem = row_win - full * _CHUNK
        if rem:
            pltpu.sync_copy(rows_v.at[0, pl.ds(0, rem)],
                            agg_sh.at[pl.ds(row0 + full * _CHUNK, rem)])
        plsc.subcore_barrier()

        def load_idx(t, islot):
            return pltpu.make_async_copy(
                ei_hbm.at[:, pl.ds((c0 + t) * _CHUNK, _CHUNK)],
                idx_v.at[islot], isem.at[islot])

        def gather(bslot, islot):
            return pltpu.make_async_copy(x_hbm.at[idx_v.at[islot, 0]],
                                         rows_v.at[bslot], gsem.at[bslot])

        def scatter(bslot, islot):
            return pltpu.make_async_copy(rows_v.at[bslot],
                                         agg_sh.at[idx_v.at[islot, 1]],
                                         ssem.at[bslot])

        for j in range(_NI // 2):
            load_idx(j, j).start()
        load_idx(0, 0).wait()
        gather(0, 0).start()

        def group(g, carry):
            for u in range(_NI):
                t = g * _NI + u
                b = u % _NB
                ib = u % _NI

                @pl.when(t < n_my)
                def _():
                    gather(b, ib).wait()
                    scatter(b, ib).start(add=True)
                t1 = t + 1
                b1 = (u + 1) % _NB
                ib1 = (u + 1) % _NI

                @pl.when(jnp.logical_and(t1 < n_my, t1 >= _NB))
                def _():
                    scatter(b1, ib1).wait()
                t3 = t + _NI // 2
                ib3 = (u + _NI // 2) % _NI

                @pl.when(t3 < n_my)
                def _():
                    load_idx(t3, ib3).start()

                @pl.when(t1 < n_my)
                def _():
                    load_idx(t1, ib1).wait()
                    gather(b1, ib1).start()
            return carry

        lax.fori_loop(0, (n_my + _NI - 1) // _NI, group, 0)
        for b in range(_NB):
            scatter(b, b).wait()
        plsc.subcore_barrier()

        @pl.when(cid == 0)
        def _():
            pltpu.sync_copy(agg_sh.at[pl.ds(row0, row_win)],
                            out0_hbm.at[pl.ds(row0, row_win)])

        @pl.when(cid == 1)
        def _():
            pltpu.sync_copy(agg_sh.at[pl.ds(row0, row_win)],
                            out1_hbm.at[pl.ds(row0, row_win)])

    return seg_kernel(x, edge_index)


_BLK = 2000


def _mlp_layer(x, p0, p1, w1, b1, w2, b2, g, bt):
    n, d = x.shape
    assert n % _BLK == 0
    inv_std = float(1.0 / (1.0 + 1e-5) ** 0.5)

    def body(x_ref, p0_ref, p1_ref, w1_ref, b1_ref, w2_ref, b2_ref, g_ref,
             bt_ref, o_ref):
        h = x_ref[...] + p0_ref[...] + p1_ref[...]
        h = lax.dot(h, w1_ref[...],
                    preferred_element_type=jnp.float32) + b1_ref[...]
        h = jnp.maximum(h, 0.0)
        h = lax.dot(h, w2_ref[...],
                    preferred_element_type=jnp.float32) + b2_ref[...]
        h = jnp.maximum(h, 0.0)
        o_ref[...] = g_ref[...] * (h * inv_std) + bt_ref[...]

    blk = pl.BlockSpec((_BLK, d), lambda i: (i, 0))
    wblk = pl.BlockSpec((d, d), lambda i: (0, 0))
    vblk = pl.BlockSpec((1, d), lambda i: (0, 0))
    return pl.pallas_call(
        body,
        grid=(n // _BLK,),
        in_specs=[blk, blk, blk, wblk, vblk, wblk, vblk, vblk, vblk],
        out_specs=blk,
        out_shape=jax.ShapeDtypeStruct((n, d), jnp.float32),
    )(x, p0, p1, w1, b1.reshape(1, d), w2, b2.reshape(1, d),
      g.reshape(1, d), bt.reshape(1, d))


def kernel(x, edge_index, W1, b1, W2, b2, gamma, beta):
    num_layers = W1.shape[0]
    out = x
    recs = []
    for i in range(num_layers):
        p0, p1 = _segment_sum_partials(out, edge_index)
        out = _mlp_layer(out, p0, p1, W1[i], b1[i], W2[i], b2[i],
                         gamma[i], beta[i])
        recs.append(out)
    return jnp.concatenate(recs, axis=-1)


# speedup vs baseline: 10.5657x; 1.0040x over previous
import functools

import jax
import jax.numpy as jnp
from jax import lax
from jax.experimental import pallas as pl
from jax.experimental.pallas import tpu as pltpu
from jax.experimental.pallas import tpu_sc as plsc

_NC = 2
_NS = 16
_NW = _NC * _NS
_CHUNK = 128
_NB = 3
_NI = 6


def _segment_sum_partials(x, edge_index):
    n, d = x.shape
    e = edge_index.shape[1]
    assert e % _CHUNK == 0 and d % 16 == 0
    n_chunks = e // _CHUNK
    row_step = (n // _NS) // 8 * 8
    row_win = n - row_step * (_NS - 1)
    assert row_win % 8 == 0 and row_win >= row_step

    mesh = plsc.VectorSubcoreMesh(core_axis_name="c", subcore_axis_name="s")

    @functools.partial(
        pl.kernel,
        mesh=mesh,
        out_type=[jax.ShapeDtypeStruct((n, d), jnp.float32),
                  jax.ShapeDtypeStruct((n, d), jnp.float32)],
        scratch_types=[
            pltpu.VMEM((_NI, 2, _CHUNK), jnp.int32),
            pltpu.VMEM((_NB, _CHUNK, d), jnp.float32),
            pltpu.VMEM_SHARED((n, d), jnp.float32),
            pltpu.SemaphoreType.DMA((_NI,)),
            pltpu.SemaphoreType.DMA((_NB,)),
            pltpu.SemaphoreType.DMA((_NB,)),
        ],
    )
    def seg_kernel(x_hbm, ei_hbm, out0_hbm, out1_hbm, idx_v, rows_v, agg_sh,
                   isem, gsem, ssem):
        cid = lax.axis_index("c")
        sid = lax.axis_index("s")
        wid = sid * _NC + cid

        zvec = jnp.zeros((16,), jnp.float32)

        def zero_row(r, carry):
            for c in range(d // 16):
                rows_v[0, r, pl.ds(c * 16, 16)] = zvec
            return carry

        c0 = (n_chunks * wid) // _NW
        n_my = (n_chunks * (wid + 1)) // _NW - c0

        lax.fori_loop(0, _CHUNK, zero_row, 0)
        row0 = sid * row_step
        full = row_win // _CHUNK
        for k in range(full):
            pltpu.sync_copy(rows_v.at[0],
                            agg_sh.at[pl.ds(row0 + k * _CHUNK, _CHUNK)])
        rem = row_win - full * _CHUNK
        if rem:
            pltpu.sync_copy(rows_v.at[0, pl.ds(0, rem)],
                            agg_sh.at[pl.ds(row0 + full * _CHUNK, rem)])
        plsc.subcore_barrier()

        def load_idx(t, islot):
            return pltpu.make_async_copy(
                ei_hbm.at[:, pl.ds((c0 + t) * _CHUNK, _CHUNK)],
                idx_v.at[islot], isem.at[islot])

        def gather(bslot, islot):
            return pltpu.make_async_copy(x_hbm.at[idx_v.at[islot, 0]],
                                         rows_v.at[bslot], gsem.at[bslot])

        def scatter(bslot, islot):
            return pltpu.make_async_copy(rows_v.at[bslot],
                                         agg_sh.at[idx_v.at[islot, 1]],
                                         ssem.at[bslot])

        for j in range(_NI // 2):
            load_idx(j, j).start()
        load_idx(0, 0).wait()
        gather(0, 0).start()

        def group(g, carry):
            for u in range(_NI):
                t = g * _NI + u
                b = u % _NB
                ib = u % _NI

                t1 = t + 1
                b1 = (u + 1) % _NB
                ib1 = (u + 1) % _NI

                @pl.when(t < n_my)
                def _():
                    gather(b, ib).wait()

                @pl.when(jnp.logical_and(t1 < n_my, t1 >= _NB))
                def _():
                    scatter(b1, ib1).wait()

                @pl.when(t1 < n_my)
                def _():
                    load_idx(t1, ib1).wait()
                    gather(b1, ib1).start()

                @pl.when(t < n_my)
                def _():
                    scatter(b, ib).start(add=True)
                t3 = t + _NI // 2
                ib3 = (u + _NI // 2) % _NI

                @pl.when(t3 < n_my)
                def _():
                    load_idx(t3, ib3).start()
            return carry

        lax.fori_loop(0, (n_my + _NI - 1) // _NI, group, 0)
        for b in range(_NB):
            scatter(b, b).wait()
        plsc.subcore_barrier()

        @pl.when(cid == 0)
        def _():
            pltpu.sync_copy(agg_sh.at[pl.ds(row0, row_win)],
                            out0_hbm.at[pl.ds(row0, row_win)])

        @pl.when(cid == 1)
        def _():
            pltpu.sync_copy(agg_sh.at[pl.ds(row0, row_win)],
                            out1_hbm.at[pl.ds(row0, row_win)])

    return seg_kernel(x, edge_index)


_BLK = 2000


def _mlp_layer(x, p0, p1, w1, b1, w2, b2, g, bt):
    n, d = x.shape
    assert n % _BLK == 0
    inv_std = float(1.0 / (1.0 + 1e-5) ** 0.5)

    def body(x_ref, p0_ref, p1_ref, w1_ref, b1_ref, w2_ref, b2_ref, g_ref,
             bt_ref, o_ref):
        h = x_ref[...] + p0_ref[...] + p1_ref[...]
        h = lax.dot(h, w1_ref[...],
                    preferred_element_type=jnp.float32) + b1_ref[...]
        h = jnp.maximum(h, 0.0)
        h = lax.dot(h, w2_ref[...],
                    preferred_element_type=jnp.float32) + b2_ref[...]
        h = jnp.maximum(h, 0.0)
        o_ref[...] = g_ref[...] * (h * inv_std) + bt_ref[...]

    blk = pl.BlockSpec((_BLK, d), lambda i: (i, 0))
    wblk = pl.BlockSpec((d, d), lambda i: (0, 0))
    vblk = pl.BlockSpec((1, d), lambda i: (0, 0))
    return pl.pallas_call(
        body,
        grid=(n // _BLK,),
        in_specs=[blk, blk, blk, wblk, vblk, wblk, vblk, vblk, vblk],
        out_specs=blk,
        out_shape=jax.ShapeDtypeStruct((n, d), jnp.float32),
    )(x, p0, p1, w1, b1.reshape(1, d), w2, b2.reshape(1, d),
      g.reshape(1, d), bt.reshape(1, d))


def kernel(x, edge_index, W1, b1, W2, b2, gamma, beta):
    num_layers = W1.shape[0]
    out = x
    recs = []
    for i in range(num_layers):
        p0, p1 = _segment_sum_partials(out, edge_index)
        out = _mlp_layer(out, p0, p1, W1[i], b1[i], W2[i], b2[i],
                         gamma[i], beta[i])
        recs.append(out)
    return jnp.concatenate(recs, axis=-1)
